# baseline (speedup 1.0000x reference)
import functools

import jax
import jax.numpy as jnp
from jax import lax
from jax.experimental import pallas as pl
from jax.experimental.pallas import tpu as pltpu
from jax.experimental.pallas import tpu_sc as plsc

B, N, E = 256, 192, 3072
C_IN = 128
GB = 32

NSC, NTILES = 2, 16
EPT = E // (NSC * NTILES)
CSLICE = (N * N) // NTILES


def _sc_counts_body(ei_hbm, out_hbm, s_v, d_v, idx_v, ones_v, zero_v, c_sh):
    cid = lax.axis_index("c")
    sid = lax.axis_index("s")
    goff = (cid * NTILES + sid) * EPT
    pltpu.sync_copy(ei_hbm.at[pl.ds(goff, EPT)], s_v)
    pltpu.sync_copy(ei_hbm.at[pl.ds(E + goff, EPT)], d_v)
    for j in range(EPT // 16):
        sl = pl.ds(16 * j, 16)
        idx_v[sl] = d_v[sl] * N + s_v[sl]
        ones_v[sl] = jnp.full((16,), 1.0, jnp.float32)

    def _z(i, carry):
        zero_v[pl.ds(i * 16, 16)] = jnp.zeros((16,), jnp.float32)
        return carry

    lax.fori_loop(0, CSLICE // 16, _z, 0)
    pltpu.sync_copy(zero_v, c_sh.at[pl.ds(sid * CSLICE, CSLICE)])
    plsc.subcore_barrier()
    pltpu.sync_copy(ones_v, c_sh.at[idx_v], add=True)
    plsc.subcore_barrier()
    pltpu.sync_copy(c_sh.at[pl.ds(sid * CSLICE, CSLICE)],
                    out_hbm.at[pl.ds(cid * (N * N) + sid * CSLICE, CSLICE)])


def _sc_counts(edge_index):
    mesh = plsc.VectorSubcoreMesh(core_axis_name="c", subcore_axis_name="s")
    run = functools.partial(
        pl.kernel,
        out_type=jax.ShapeDtypeStruct((NSC * N * N,), jnp.float32),
        mesh=mesh,
        scratch_types=[
            pltpu.VMEM((EPT,), jnp.int32),
            pltpu.VMEM((EPT,), jnp.int32),
            pltpu.VMEM((EPT,), jnp.int32),
            pltpu.VMEM((EPT,), jnp.float32),
            pltpu.VMEM((CSLICE,), jnp.float32),
            pltpu.VMEM_SHARED((N * N,), jnp.float32),
        ],
    )(_sc_counts_body)
    return run(edge_index.reshape(2 * E).astype(jnp.int32))


def _conv_s2(X, w_ref, b_ref, K, pad):
    T = X.shape[0] // 2
    gb, Cin = X.shape[1], X.shape[2]
    Cout = w_ref.shape[2]
    Xp = X.reshape(T, 2, gb, Cin)
    z = jnp.zeros((2, 2, gb, Cin), X.dtype)
    Xpad = jnp.concatenate([z, Xp, z], axis=0)
    acc = None
    for k in range(K):
        j = k - pad
        m, r = j // 2, j % 2
        sl = Xpad[2 + m:2 + m + T, r]
        term = jnp.dot(sl.reshape(T * gb, Cin), w_ref[k],
                       preferred_element_type=jnp.float32)
        acc = term if acc is None else acc + term
    acc = acc.reshape(T, gb, Cout) + b_ref[...].reshape(1, 1, Cout)
    return jnp.maximum(acc, 0.0)


def _main_body(src_ref, c_ref, w1_ref, b1t_ref, w2_ref, b2t_ref,
               cw1_ref, cb1_ref, cw2_ref, cb2_ref, cw3_ref, cb3_ref,
               out_ref, a2_s, s1t, s2t, x4_s):
    i = pl.program_id(0)

    @pl.when(i == 0)
    def _():
        C = c_ref[0] + c_ref[1]
        r = lax.broadcasted_iota(jnp.int32, (N, N), 0)
        c = lax.broadcasted_iota(jnp.int32, (N, N), 1)
        C = C + (r == c).astype(jnp.float32)
        CT = C.T
        deg_col = jnp.sum(C, axis=1, keepdims=True)
        deg_row = jnp.sum(CT, axis=0, keepdims=True)
        a2_s[...] = CT * lax.rsqrt(deg_col) * lax.rsqrt(deg_row)

    A2 = a2_s[...]
    w1 = w1_ref[...]
    w2 = w2_ref[...]
    c0 = (((0,), (0,)), ((), ()))
    for b in range(GB):
        s1t[64 * b:64 * (b + 1), :] = jax.lax.dot_general(
            w1, src_ref[b], c0, preferred_element_type=jnp.float32)
    s1t[...] = jnp.maximum(
        jnp.dot(s1t[...], A2, preferred_element_type=jnp.float32)
        + b1t_ref[...], 0.0)
    for b in range(GB):
        s2t[32 * b:32 * (b + 1), :] = jax.lax.dot_general(
            w2, s1t[64 * b:64 * (b + 1), :], c0,
            preferred_element_type=jnp.float32)
    s2t[...] = jnp.maximum(
        jnp.dot(s2t[...], A2, preferred_element_type=jnp.float32)
        + b2t_ref[...], 0.0)
    ey = (lax.broadcasted_iota(jnp.int32, (32, 32), 0)
          == lax.broadcasted_iota(jnp.int32, (32, 32), 1)).astype(jnp.float32)
    for b in range(GB):
        x4_s[:, b, :] = jax.lax.dot_general(
            s2t[32 * b:32 * (b + 1), :], ey, (((0,), (0,)), ((), ())),
            preferred_element_type=jnp.float32)
    x4 = x4_s[...]
    y = _conv_s2(x4, cw1_ref, cb1_ref, K=7, pad=3)
    y = _conv_s2(y, cw2_ref, cb2_ref, K=5, pad=2)
    y = _conv_s2(y, cw3_ref, cb3_ref, K=3, pad=1)
    out_ref[...] = y


def _run_main(src, counts, W_g1, b_g1, W_g2, b_g2, cw1, cb1, cw2, cb2,
              cw3, cb3):
    nsteps = B // GB
    full = lambda shape: pl.BlockSpec(shape, lambda i: (0,) * len(shape))
    return pl.pallas_call(
        _main_body,
        grid=(nsteps,),
        in_specs=[
            pl.BlockSpec((GB, C_IN, N), lambda i: (i, 0, 0)),
            full((NSC, N, N)),
            full((C_IN, 64)), full((GB * 64, 1)),
            full((64, 32)), full((GB * 32, 1)),
            full((7, 32, 32)), full((1, 32)),
            full((5, 32, 64)), full((1, 64)),
            full((3, 64, 128)), full((1, 128)),
        ],
        out_specs=pl.BlockSpec((24, GB, 128), lambda i: (0, i, 0)),
        out_shape=jax.ShapeDtypeStruct((24, B, 128), jnp.float32),
        scratch_shapes=[pltpu.VMEM((N, N), jnp.float32),
                        pltpu.VMEM((GB * 64, N), jnp.float32),
                        pltpu.VMEM((GB * 32, N), jnp.float32),
                        pltpu.VMEM((N, GB, 32), jnp.float32)],
    )(src, counts, W_g1, jnp.tile(b_g1, GB).reshape(GB * 64, 1),
      W_g2, jnp.tile(b_g2, GB).reshape(GB * 32, 1),
      cw1, cb1.reshape(1, 32), cw2, cb2.reshape(1, 64), cw3, cb3.reshape(1, 128))


def _mlp_body(f_ref, wr_ref, fb_ref, c1w_ref, c1b_ref, c2w_ref, c2b_ref,
              logits_ref, feat_ref):
    acc = jnp.zeros((B, 256), jnp.float32)
    for t in range(24):
        acc = acc + jnp.dot(f_ref[t], wr_ref[t],
                            preferred_element_type=jnp.float32)
    feat = acc + fb_ref[...]
    feat_ref[...] = feat
    h = jnp.maximum(feat, 0.0)
    h = jnp.maximum(jnp.dot(h, c1w_ref[...],
                            preferred_element_type=jnp.float32) + c1b_ref[...], 0.0)
    logits_ref[...] = jnp.dot(h, c2w_ref[...],
                              preferred_element_type=jnp.float32) + c2b_ref[...]


def _run_mlp(feats, wr, fc1_b, cls1_w, cls1_b, cls2_w, cls2_b):
    return pl.pallas_call(
        _mlp_body,
        out_shape=(jax.ShapeDtypeStruct((B, 210), jnp.float32),
                   jax.ShapeDtypeStruct((B, 256), jnp.float32)),
    )(feats, wr, fc1_b.reshape(1, 256), cls1_w, cls1_b.reshape(1, 256),
      cls2_w, cls2_b.reshape(1, 210))


def kernel(src, edge_index, W_g1, b_g1, W_g2, b_g2, conv1_w, conv1_b,
           conv2_w, conv2_b, conv3_w, conv3_b, fc1_w, fc1_b,
           cls1_w, cls1_b, cls2_w, cls2_b):
    counts = _sc_counts(edge_index).reshape(NSC, N, N)
    cw1 = conv1_w.transpose(2, 1, 0)
    cw2 = conv2_w.transpose(2, 1, 0)
    cw3 = conv3_w.transpose(2, 1, 0)
    wr = fc1_w.reshape(128, 24, 256).transpose(1, 0, 2)
    feats = _run_main(src, counts, W_g1, b_g1, W_g2, b_g2,
                      cw1, conv1_b, cw2, conv2_b, cw3, conv3_b)
    logits, feat = _run_mlp(feats, wr, fc1_b, cls1_w, cls1_b, cls2_w, cls2_b)
    return (logits, feat)


# speedup vs baseline: 1.0339x; 1.0339x over previous
import functools

import jax
import jax.numpy as jnp
from jax import lax
from jax.experimental import pallas as pl
from jax.experimental.pallas import tpu as pltpu
from jax.experimental.pallas import tpu_sc as plsc

B, N, E = 256, 192, 3072
C_IN = 128
GB = 32

NSC, NTILES = 2, 16
EPT = E // (NSC * NTILES)
CSLICE = (N * N) // NTILES


def _sc_counts_body(ei_hbm, out_hbm, s_v, d_v, idx_v, ones_v, zero_v, c_sh):
    cid = lax.axis_index("c")
    sid = lax.axis_index("s")
    goff = (cid * NTILES + sid) * EPT
    pltpu.sync_copy(ei_hbm.at[pl.ds(goff, EPT)], s_v)
    pltpu.sync_copy(ei_hbm.at[pl.ds(E + goff, EPT)], d_v)
    for j in range(EPT // 16):
        sl = pl.ds(16 * j, 16)
        idx_v[sl] = d_v[sl] * N + s_v[sl]
        ones_v[sl] = jnp.full((16,), 1.0, jnp.float32)

    def _z(i, carry):
        zero_v[pl.ds(i * 16, 16)] = jnp.zeros((16,), jnp.float32)
        return carry

    lax.fori_loop(0, CSLICE // 16, _z, 0)
    pltpu.sync_copy(zero_v, c_sh.at[pl.ds(sid * CSLICE, CSLICE)])
    plsc.subcore_barrier()
    pltpu.sync_copy(ones_v, c_sh.at[idx_v], add=True)
    plsc.subcore_barrier()
    pltpu.sync_copy(c_sh.at[pl.ds(sid * CSLICE, CSLICE)],
                    out_hbm.at[pl.ds(cid * (N * N) + sid * CSLICE, CSLICE)])


def _sc_counts(edge_index):
    mesh = plsc.VectorSubcoreMesh(core_axis_name="c", subcore_axis_name="s")
    run = functools.partial(
        pl.kernel,
        out_type=jax.ShapeDtypeStruct((NSC * N * N,), jnp.float32),
        mesh=mesh,
        scratch_types=[
            pltpu.VMEM((EPT,), jnp.int32),
            pltpu.VMEM((EPT,), jnp.int32),
            pltpu.VMEM((EPT,), jnp.int32),
            pltpu.VMEM((EPT,), jnp.float32),
            pltpu.VMEM((CSLICE,), jnp.float32),
            pltpu.VMEM_SHARED((N * N,), jnp.float32),
        ],
    )(_sc_counts_body)
    return run(edge_index.reshape(2 * E).astype(jnp.int32))


def _conv_s2(X, w_ref, b_ref, K, pad):
    T = X.shape[0] // 2
    gb, Cin = X.shape[1], X.shape[2]
    Cout = w_ref.shape[2]
    Xp = X.reshape(T, 2, gb, Cin)
    z = jnp.zeros((2, 2, gb, Cin), X.dtype)
    Xpad = jnp.concatenate([z, Xp, z], axis=0)
    acc = None
    for k in range(K):
        j = k - pad
        m, r = j // 2, j % 2
        sl = Xpad[2 + m:2 + m + T, r]
        term = jnp.dot(sl.reshape(T * gb, Cin), w_ref[k],
                       preferred_element_type=jnp.float32)
        acc = term if acc is None else acc + term
    acc = acc.reshape(T, gb, Cout) + b_ref[...].reshape(1, 1, Cout)
    return jnp.maximum(acc, 0.0)


def _main_body(src_ref, c_ref, w1_ref, b1t_ref, w2_ref, b2t_ref,
               cw1_ref, cb1_ref, cw2_ref, cb2_ref, cw3_ref, cb3_ref,
               out_ref, a2_s, s1t, s2t, x4_s):
    i = pl.program_id(0)

    @pl.when(i == 0)
    def _():
        C = c_ref[0] + c_ref[1]
        r = lax.broadcasted_iota(jnp.int32, (N, N), 0)
        c = lax.broadcasted_iota(jnp.int32, (N, N), 1)
        C = C + (r == c).astype(jnp.float32)
        CT = C.T
        deg_col = jnp.sum(C, axis=1, keepdims=True)
        deg_row = jnp.sum(CT, axis=0, keepdims=True)
        a2_s[...] = CT * lax.rsqrt(deg_col) * lax.rsqrt(deg_row)

    A2 = a2_s[...]
    w1 = w1_ref[...]
    w2 = w2_ref[...]
    c0 = (((0,), (0,)), ((), ()))
    for b in range(GB):
        s1t[64 * b:64 * (b + 1), :] = jax.lax.dot_general(
            w1, src_ref[b], c0, preferred_element_type=jnp.float32)
    s1t[...] = jnp.maximum(
        jnp.dot(s1t[...], A2, preferred_element_type=jnp.float32)
        + b1t_ref[...], 0.0)
    for b in range(GB):
        s2t[32 * b:32 * (b + 1), :] = jax.lax.dot_general(
            w2, s1t[64 * b:64 * (b + 1), :], c0,
            preferred_element_type=jnp.float32)
    s2t[...] = jnp.maximum(
        jnp.dot(s2t[...], A2, preferred_element_type=jnp.float32)
        + b2t_ref[...], 0.0)
    for b in range(GB):
        x4_s[:, b, :] = s2t[32 * b:32 * (b + 1), :].T
    x4 = x4_s[...]
    y = _conv_s2(x4, cw1_ref, cb1_ref, K=7, pad=3)
    y = _conv_s2(y, cw2_ref, cb2_ref, K=5, pad=2)
    y = _conv_s2(y, cw3_ref, cb3_ref, K=3, pad=1)
    out_ref[...] = y


def _run_main(src, counts, W_g1, b_g1, W_g2, b_g2, cw1, cb1, cw2, cb2,
              cw3, cb3):
    nsteps = B // GB
    full = lambda shape: pl.BlockSpec(shape, lambda i: (0,) * len(shape))
    return pl.pallas_call(
        _main_body,
        grid=(nsteps,),
        in_specs=[
            pl.BlockSpec((GB, C_IN, N), lambda i: (i, 0, 0)),
            full((NSC, N, N)),
            full((C_IN, 64)), full((GB * 64, 1)),
            full((64, 32)), full((GB * 32, 1)),
            full((7, 32, 32)), full((1, 32)),
            full((5, 32, 64)), full((1, 64)),
            full((3, 64, 128)), full((1, 128)),
        ],
        out_specs=pl.BlockSpec((24, GB, 128), lambda i: (0, i, 0)),
        out_shape=jax.ShapeDtypeStruct((24, B, 128), jnp.float32),
        scratch_shapes=[pltpu.VMEM((N, N), jnp.float32),
                        pltpu.VMEM((GB * 64, N), jnp.float32),
                        pltpu.VMEM((GB * 32, N), jnp.float32),
                        pltpu.VMEM((N, GB, 32), jnp.float32)],
    )(src, counts, W_g1, jnp.tile(b_g1, GB).reshape(GB * 64, 1),
      W_g2, jnp.tile(b_g2, GB).reshape(GB * 32, 1),
      cw1, cb1.reshape(1, 32), cw2, cb2.reshape(1, 64), cw3, cb3.reshape(1, 128))


def _mlp_body(f_ref, wr_ref, fb_ref, c1w_ref, c1b_ref, c2w_ref, c2b_ref,
              logits_ref, feat_ref):
    acc = jnp.zeros((B, 256), jnp.float32)
    for t in range(24):
        acc = acc + jnp.dot(f_ref[t], wr_ref[t],
                            preferred_element_type=jnp.float32)
    feat = acc + fb_ref[...]
    feat_ref[...] = feat
    h = jnp.maximum(feat, 0.0)
    h = jnp.maximum(jnp.dot(h, c1w_ref[...],
                            preferred_element_type=jnp.float32) + c1b_ref[...], 0.0)
    logits_ref[...] = jnp.dot(h, c2w_ref[...],
                              preferred_element_type=jnp.float32) + c2b_ref[...]


def _run_mlp(feats, wr, fc1_b, cls1_w, cls1_b, cls2_w, cls2_b):
    return pl.pallas_call(
        _mlp_body,
        out_shape=(jax.ShapeDtypeStruct((B, 210), jnp.float32),
                   jax.ShapeDtypeStruct((B, 256), jnp.float32)),
    )(feats, wr, fc1_b.reshape(1, 256), cls1_w, cls1_b.reshape(1, 256),
      cls2_w, cls2_b.reshape(1, 210))


def kernel(src, edge_index, W_g1, b_g1, W_g2, b_g2, conv1_w, conv1_b,
           conv2_w, conv2_b, conv3_w, conv3_b, fc1_w, fc1_b,
           cls1_w, cls1_b, cls2_w, cls2_b):
    counts = _sc_counts(edge_index).reshape(NSC, N, N)
    cw1 = conv1_w.transpose(2, 1, 0)
    cw2 = conv2_w.transpose(2, 1, 0)
    cw3 = conv3_w.transpose(2, 1, 0)
    wr = fc1_w.reshape(128, 24, 256).transpose(1, 0, 2)
    feats = _run_main(src, counts, W_g1, b_g1, W_g2, b_g2,
                      cw1, conv1_b, cw2, conv2_b, cw3, conv3_b)
    logits, feat = _run_mlp(feats, wr, fc1_b, cls1_w, cls1_b, cls2_w, cls2_b)
    return (logits, feat)


# speedup vs baseline: 1.2347x; 1.1943x over previous
import functools

import jax
import jax.numpy as jnp
from jax import lax
from jax.experimental import pallas as pl
from jax.experimental.pallas import tpu as pltpu
from jax.experimental.pallas import tpu_sc as plsc

B, N, E = 256, 192, 3072
C_IN = 128
GB = 32

NSC, NTILES = 2, 16
EPT = E // (NSC * NTILES)
CSLICE = (N * N) // NTILES


def _sc_counts_body(ei_hbm, out_hbm, s_v, d_v, idx_v, ones_v, zero_v, c_sh):
    cid = lax.axis_index("c")
    sid = lax.axis_index("s")
    goff = (cid * NTILES + sid) * EPT
    pltpu.sync_copy(ei_hbm.at[pl.ds(goff, EPT)], s_v)
    pltpu.sync_copy(ei_hbm.at[pl.ds(E + goff, EPT)], d_v)
    for j in range(EPT // 16):
        sl = pl.ds(16 * j, 16)
        idx_v[sl] = d_v[sl] * N + s_v[sl]
        ones_v[sl] = jnp.full((16,), 1.0, jnp.float32)

    def _z(i, carry):
        zero_v[pl.ds(i * 16, 16)] = jnp.zeros((16,), jnp.float32)
        return carry

    lax.fori_loop(0, CSLICE // 16, _z, 0)
    pltpu.sync_copy(zero_v, c_sh.at[pl.ds(sid * CSLICE, CSLICE)])
    plsc.subcore_barrier()
    pltpu.sync_copy(ones_v, c_sh.at[idx_v], add=True)
    plsc.subcore_barrier()
    pltpu.sync_copy(c_sh.at[pl.ds(sid * CSLICE, CSLICE)],
                    out_hbm.at[pl.ds(cid * (N * N) + sid * CSLICE, CSLICE)])


def _sc_counts(edge_index):
    mesh = plsc.VectorSubcoreMesh(core_axis_name="c", subcore_axis_name="s")
    run = functools.partial(
        pl.kernel,
        out_type=jax.ShapeDtypeStruct((NSC * N * N,), jnp.float32),
        mesh=mesh,
        scratch_types=[
            pltpu.VMEM((EPT,), jnp.int32),
            pltpu.VMEM((EPT,), jnp.int32),
            pltpu.VMEM((EPT,), jnp.int32),
            pltpu.VMEM((EPT,), jnp.float32),
            pltpu.VMEM((CSLICE,), jnp.float32),
            pltpu.VMEM_SHARED((N * N,), jnp.float32),
        ],
    )(_sc_counts_body)
    return run(edge_index.reshape(2 * E).astype(jnp.int32))


def _conv_s2(X, w_ref, b_ref, K, pad):
    T = X.shape[0] // 2
    gb, Cin = X.shape[1], X.shape[2]
    Cout = w_ref.shape[2]
    Xp = X.reshape(T, 2, gb, Cin)
    z = jnp.zeros((2, 2, gb, Cin), X.dtype)
    Xpad = jnp.concatenate([z, Xp, z], axis=0)
    acc = None
    for k in range(K):
        j = k - pad
        m, r = j // 2, j % 2
        sl = Xpad[2 + m:2 + m + T, r]
        term = jnp.dot(sl.reshape(T * gb, Cin), w_ref[k],
                       preferred_element_type=jnp.float32)
        acc = term if acc is None else acc + term
    acc = acc.reshape(T, gb, Cout) + b_ref[...].reshape(1, 1, Cout)
    return jnp.maximum(acc, 0.0)


def _main_body(src_ref, c_ref, w1t_ref, b1t_ref, w2t_ref, b2t_ref,
               cw1_ref, cb1_ref, cw2_ref, cb2_ref, cw3_ref, cb3_ref,
               out_ref, a2_s, s1t, s2t, x4_s):
    i = pl.program_id(0)

    @pl.when(i == 0)
    def _():
        C = c_ref[0] + c_ref[1]
        r = lax.broadcasted_iota(jnp.int32, (N, N), 0)
        c = lax.broadcasted_iota(jnp.int32, (N, N), 1)
        C = C + (r == c).astype(jnp.float32)
        CT = C.T
        deg_col = jnp.sum(C, axis=1, keepdims=True)
        deg_row = jnp.sum(CT, axis=0, keepdims=True)
        a2_s[...] = CT * lax.rsqrt(deg_col) * lax.rsqrt(deg_row)

    A2 = a2_s[...]
    w1t = w1t_ref[...]
    w2t = w2t_ref[...]
    for b in range(GB):
        s1t[64 * b:64 * (b + 1), :] = jax.lax.dot_general(
            w1t, src_ref[b], (((1,), (1,)), ((), ())),
            preferred_element_type=jnp.float32)
    s1t[...] = jnp.maximum(
        jnp.dot(s1t[...], A2, preferred_element_type=jnp.float32)
        + b1t_ref[...], 0.0)
    for b in range(GB):
        s2t[32 * b:32 * (b + 1), :] = jax.lax.dot_general(
            w2t, s1t[64 * b:64 * (b + 1), :], (((1,), (0,)), ((), ())),
            preferred_element_type=jnp.float32)
    s2t[...] = jnp.maximum(
        jnp.dot(s2t[...], A2, preferred_element_type=jnp.float32)
        + b2t_ref[...], 0.0)
    for b in range(GB):
        x4_s[:, b, :] = s2t[32 * b:32 * (b + 1), :].T
    x4 = x4_s[...]
    y = _conv_s2(x4, cw1_ref, cb1_ref, K=7, pad=3)
    y = _conv_s2(y, cw2_ref, cb2_ref, K=5, pad=2)
    y = _conv_s2(y, cw3_ref, cb3_ref, K=3, pad=1)
    out_ref[...] = y


def _run_main(srcT, counts, w1t, b_g1, w2t, b_g2, cw1, cb1, cw2, cb2,
              cw3, cb3):
    nsteps = B // GB
    full = lambda shape: pl.BlockSpec(shape, lambda i: (0,) * len(shape))
    return pl.pallas_call(
        _main_body,
        grid=(nsteps,),
        in_specs=[
            pl.BlockSpec((GB, N, C_IN), lambda i: (i, 0, 0)),
            full((NSC, N, N)),
            full((64, C_IN)), full((GB * 64, 1)),
            full((32, 64)), full((GB * 32, 1)),
            full((7, 32, 32)), full((1, 32)),
            full((5, 32, 64)), full((1, 64)),
            full((3, 64, 128)), full((1, 128)),
        ],
        out_specs=pl.BlockSpec((24, GB, 128), lambda i: (0, i, 0)),
        out_shape=jax.ShapeDtypeStruct((24, B, 128), jnp.float32),
        scratch_shapes=[pltpu.VMEM((N, N), jnp.float32),
                        pltpu.VMEM((GB * 64, N), jnp.float32),
                        pltpu.VMEM((GB * 32, N), jnp.float32),
                        pltpu.VMEM((N, GB, 32), jnp.float32)],
    )(srcT, counts, w1t, jnp.tile(b_g1, GB).reshape(GB * 64, 1),
      w2t, jnp.tile(b_g2, GB).reshape(GB * 32, 1),
      cw1, cb1.reshape(1, 32), cw2, cb2.reshape(1, 64), cw3, cb3.reshape(1, 128))


def _mlp_body(f_ref, wr_ref, fb_ref, c1wt_ref, c1b_ref, c2wt_ref, c2b_ref,
              logitsT_ref, featT_ref):
    acc = jnp.zeros((B, 256), jnp.float32)
    for t in range(24):
        acc = acc + jnp.dot(f_ref[t], wr_ref[t],
                            preferred_element_type=jnp.float32)
    feat = acc + fb_ref[...]
    featT = feat.T
    featT_ref[...] = featT
    hT = jnp.maximum(featT, 0.0)
    hT = jnp.maximum(jnp.dot(c1wt_ref[...], hT,
                             preferred_element_type=jnp.float32)
                     + c1b_ref[...], 0.0)
    logitsT_ref[...] = jnp.dot(c2wt_ref[...], hT,
                               preferred_element_type=jnp.float32) + c2b_ref[...]


def _run_mlp(feats, wr, fc1_b, c1wt, cls1_b, c2wt, cls2_b):
    return pl.pallas_call(
        _mlp_body,
        out_shape=(jax.ShapeDtypeStruct((210, B), jnp.float32),
                   jax.ShapeDtypeStruct((256, B), jnp.float32)),
    )(feats, wr, fc1_b.reshape(1, 256), c1wt, cls1_b.reshape(256, 1),
      c2wt, cls2_b.reshape(210, 1))


def kernel(src, edge_index, W_g1, b_g1, W_g2, b_g2, conv1_w, conv1_b,
           conv2_w, conv2_b, conv3_w, conv3_b, fc1_w, fc1_b,
           cls1_w, cls1_b, cls2_w, cls2_b):
    counts = _sc_counts(edge_index).reshape(NSC, N, N)
    srcT = jnp.transpose(src, (0, 2, 1))
    w1t = W_g1.T
    w2t = W_g2.T
    c1wt = cls1_w.T
    c2wt = cls2_w.T
    cw1 = conv1_w.transpose(2, 1, 0)
    cw2 = conv2_w.transpose(2, 1, 0)
    cw3 = conv3_w.transpose(2, 1, 0)
    wr = fc1_w.reshape(128, 24, 256).transpose(1, 0, 2)
    feats = _run_main(srcT, counts, w1t, b_g1, w2t, b_g2,
                      cw1, conv1_b, cw2, conv2_b, cw3, conv3_b)
    logitsT, featT = _run_mlp(feats, wr, fc1_b, c1wt, cls1_b, c2wt, cls2_b)
    return (logitsT.T, featT.T)


# speedup vs baseline: 1.2580x; 1.0188x over previous
import functools

import jax
import jax.numpy as jnp
from jax import lax
from jax.experimental import pallas as pl
from jax.experimental.pallas import tpu as pltpu
from jax.experimental.pallas import tpu_sc as plsc

B, N, E = 256, 192, 3072
C_IN = 128
GB = 64

NSC, NTILES = 2, 16
EPT = E // (NSC * NTILES)
CSLICE = (N * N) // NTILES


def _sc_counts_body(ei_hbm, out_hbm, s_v, d_v, idx_v, ones_v, zero_v, c_sh):
    cid = lax.axis_index("c")
    sid = lax.axis_index("s")
    goff = (cid * NTILES + sid) * EPT
    pltpu.sync_copy(ei_hbm.at[pl.ds(goff, EPT)], s_v)
    pltpu.sync_copy(ei_hbm.at[pl.ds(E + goff, EPT)], d_v)
    for j in range(EPT // 16):
        sl = pl.ds(16 * j, 16)
        idx_v[sl] = d_v[sl] * N + s_v[sl]
        ones_v[sl] = jnp.full((16,), 1.0, jnp.float32)

    def _z(i, carry):
        zero_v[pl.ds(i * 16, 16)] = jnp.zeros((16,), jnp.float32)
        return carry

    lax.fori_loop(0, CSLICE // 16, _z, 0)
    pltpu.sync_copy(zero_v, c_sh.at[pl.ds(sid * CSLICE, CSLICE)])
    plsc.subcore_barrier()
    pltpu.sync_copy(ones_v, c_sh.at[idx_v], add=True)
    plsc.subcore_barrier()
    pltpu.sync_copy(c_sh.at[pl.ds(sid * CSLICE, CSLICE)],
                    out_hbm.at[pl.ds(cid * (N * N) + sid * CSLICE, CSLICE)])


def _sc_counts(edge_index):
    mesh = plsc.VectorSubcoreMesh(core_axis_name="c", subcore_axis_name="s")
    run = functools.partial(
        pl.kernel,
        out_type=jax.ShapeDtypeStruct((NSC * N * N,), jnp.float32),
        mesh=mesh,
        scratch_types=[
            pltpu.VMEM((EPT,), jnp.int32),
            pltpu.VMEM((EPT,), jnp.int32),
            pltpu.VMEM((EPT,), jnp.int32),
            pltpu.VMEM((EPT,), jnp.float32),
            pltpu.VMEM((CSLICE,), jnp.float32),
            pltpu.VMEM_SHARED((N * N,), jnp.float32),
        ],
    )(_sc_counts_body)
    return run(edge_index.reshape(2 * E).astype(jnp.int32))


def _conv_s2(X, w_ref, b_ref, K, pad):
    T = X.shape[0] // 2
    gb, Cin = X.shape[1], X.shape[2]
    Cout = w_ref.shape[2]
    Xp = X.reshape(T, 2, gb, Cin)
    z = jnp.zeros((2, 2, gb, Cin), X.dtype)
    Xpad = jnp.concatenate([z, Xp, z], axis=0)
    acc = None
    for k in range(K):
        j = k - pad
        m, r = j // 2, j % 2
        sl = Xpad[2 + m:2 + m + T, r]
        term = jnp.dot(sl.reshape(T * gb, Cin), w_ref[k],
                       preferred_element_type=jnp.float32)
        acc = term if acc is None else acc + term
    acc = acc.reshape(T, gb, Cout) + b_ref[...].reshape(1, 1, Cout)
    return jnp.maximum(acc, 0.0)


def _main_body(src_ref, c_ref, w1t_ref, b1t_ref, w2t_ref, b2t_ref,
               cw1_ref, cb1_ref, cw2_ref, cb2_ref, cw3_ref, cb3_ref,
               out_ref, a2_s, s1t, s2t, x4_s):
    i = pl.program_id(0)

    @pl.when(i == 0)
    def _():
        C = c_ref[0] + c_ref[1]
        r = lax.broadcasted_iota(jnp.int32, (N, N), 0)
        c = lax.broadcasted_iota(jnp.int32, (N, N), 1)
        C = C + (r == c).astype(jnp.float32)
        CT = C.T
        deg_col = jnp.sum(C, axis=1, keepdims=True)
        deg_row = jnp.sum(CT, axis=0, keepdims=True)
        a2_s[...] = CT * lax.rsqrt(deg_col) * lax.rsqrt(deg_row)

    A2 = a2_s[...]
    w1t = w1t_ref[...]
    w2t = w2t_ref[...]
    for b in range(GB):
        s1t[64 * b:64 * (b + 1), :] = jax.lax.dot_general(
            w1t, src_ref[b], (((1,), (1,)), ((), ())),
            preferred_element_type=jnp.float32)
    s1t[...] = jnp.maximum(
        jnp.dot(s1t[...], A2, preferred_element_type=jnp.float32)
        + b1t_ref[...], 0.0)
    for b in range(GB):
        s2t[32 * b:32 * (b + 1), :] = jax.lax.dot_general(
            w2t, s1t[64 * b:64 * (b + 1), :], (((1,), (0,)), ((), ())),
            preferred_element_type=jnp.float32)
    s2t[...] = jnp.maximum(
        jnp.dot(s2t[...], A2, preferred_element_type=jnp.float32)
        + b2t_ref[...], 0.0)
    for b in range(GB):
        x4_s[:, b, :] = s2t[32 * b:32 * (b + 1), :].T
    x4 = x4_s[...]
    y = _conv_s2(x4, cw1_ref, cb1_ref, K=7, pad=3)
    y = _conv_s2(y, cw2_ref, cb2_ref, K=5, pad=2)
    y = _conv_s2(y, cw3_ref, cb3_ref, K=3, pad=1)
    out_ref[...] = y


def _run_main(srcT, counts, w1t, b_g1, w2t, b_g2, cw1, cb1, cw2, cb2,
              cw3, cb3):
    nsteps = B // GB
    full = lambda shape: pl.BlockSpec(shape, lambda i: (0,) * len(shape))
    return pl.pallas_call(
        _main_body,
        grid=(nsteps,),
        in_specs=[
            pl.BlockSpec((GB, N, C_IN), lambda i: (i, 0, 0)),
            full((NSC, N, N)),
            full((64, C_IN)), full((GB * 64, 1)),
            full((32, 64)), full((GB * 32, 1)),
            full((7, 32, 32)), full((1, 32)),
            full((5, 32, 64)), full((1, 64)),
            full((3, 64, 128)), full((1, 128)),
        ],
        out_specs=pl.BlockSpec((24, GB, 128), lambda i: (0, i, 0)),
        out_shape=jax.ShapeDtypeStruct((24, B, 128), jnp.float32),
        scratch_shapes=[pltpu.VMEM((N, N), jnp.float32),
                        pltpu.VMEM((GB * 64, N), jnp.float32),
                        pltpu.VMEM((GB * 32, N), jnp.float32),
                        pltpu.VMEM((N, GB, 32), jnp.float32)],
    )(srcT, counts, w1t, jnp.tile(b_g1, GB).reshape(GB * 64, 1),
      w2t, jnp.tile(b_g2, GB).reshape(GB * 32, 1),
      cw1, cb1.reshape(1, 32), cw2, cb2.reshape(1, 64), cw3, cb3.reshape(1, 128))


def _mlp_body(f_ref, wr_ref, fb_ref, c1wt_ref, c1b_ref, c2wt_ref, c2b_ref,
              logitsT_ref, featT_ref):
    acc = jnp.zeros((B, 256), jnp.float32)
    for t in range(24):
        acc = acc + jnp.dot(f_ref[t], wr_ref[t],
                            preferred_element_type=jnp.float32)
    feat = acc + fb_ref[...]
    featT = feat.T
    featT_ref[...] = featT
    hT = jnp.maximum(featT, 0.0)
    hT = jnp.maximum(jnp.dot(c1wt_ref[...], hT,
                             preferred_element_type=jnp.float32)
                     + c1b_ref[...], 0.0)
    logitsT_ref[...] = jnp.dot(c2wt_ref[...], hT,
                               preferred_element_type=jnp.float32) + c2b_ref[...]


def _run_mlp(feats, wr, fc1_b, c1wt, cls1_b, c2wt, cls2_b):
    return pl.pallas_call(
        _mlp_body,
        out_shape=(jax.ShapeDtypeStruct((210, B), jnp.float32),
                   jax.ShapeDtypeStruct((256, B), jnp.float32)),
    )(feats, wr, fc1_b.reshape(1, 256), c1wt, cls1_b.reshape(256, 1),
      c2wt, cls2_b.reshape(210, 1))


def kernel(src, edge_index, W_g1, b_g1, W_g2, b_g2, conv1_w, conv1_b,
           conv2_w, conv2_b, conv3_w, conv3_b, fc1_w, fc1_b,
           cls1_w, cls1_b, cls2_w, cls2_b):
    counts = _sc_counts(edge_index).reshape(NSC, N, N)
    srcT = jnp.transpose(src, (0, 2, 1))
    w1t = W_g1.T
    w2t = W_g2.T
    c1wt = cls1_w.T
    c2wt = cls2_w.T
    cw1 = conv1_w.transpose(2, 1, 0)
    cw2 = conv2_w.transpose(2, 1, 0)
    cw3 = conv3_w.transpose(2, 1, 0)
    wr = fc1_w.reshape(128, 24, 256).transpose(1, 0, 2)
    feats = _run_main(srcT, counts, w1t, b_g1, w2t, b_g2,
                      cw1, conv1_b, cw2, conv2_b, cw3, conv3_b)
    logitsT, featT = _run_mlp(feats, wr, fc1_b, c1wt, cls1_b, c2wt, cls2_b)
    return (logitsT.T, featT.T)
